# Initial kernel scaffold; baseline (speedup 1.0000x reference)
import jax
import jax.numpy as jnp
from jax.experimental import pallas as pl


def kernel(indices, emb_tables, W1, b1, W2, b2, W3, b3, W4, b4, W5, b5, W6, b6):
    raise NotImplementedError("write your pallas kernel here")


# speedup vs baseline: 16.8670x; 16.8670x over previous
import functools

import jax
import jax.numpy as jnp
from jax import lax
from jax.experimental import pallas as pl
from jax.experimental.pallas import tpu as pltpu
from jax.experimental.pallas import tpu_sc as plsc

B = 4096
F = 26
V = 1001
D = 29
DP = 32
CIN = F * DP

_NW = 32
_BPW = B // _NW


def _gather_body(table_hbm, idx_hbm, out_hbm, idx_v, rows_v, sem):
    c = lax.axis_index("c")
    s = lax.axis_index("s")
    wid = s * 2 + c
    b0 = wid * _BPW

    for f in range(F):
        pltpu.sync_copy(idx_hbm.at[pl.ds(f * B + b0, _BPW)], idx_v)
        off = f * V
        for i in range(_BPW // 16):
            sl = pl.ds(i * 16, 16)
            idx_v[sl] = idx_v[sl] + off
        pltpu.async_copy(table_hbm.at[idx_v], rows_v, sem).wait()
        pltpu.sync_copy(rows_v, out_hbm.at[pl.ds(b0, _BPW), pl.ds(f * DP, DP)])


def _sc_gather(table_flat, idx_flat):
    mesh = plsc.VectorSubcoreMesh(core_axis_name="c", subcore_axis_name="s")
    k = functools.partial(
        pl.kernel,
        mesh=mesh,
        out_type=jax.ShapeDtypeStruct((B, CIN), jnp.float32),
        scratch_types=[
            pltpu.VMEM((_BPW,), jnp.int32),
            pltpu.VMEM((_BPW, DP), jnp.float32),
            pltpu.SemaphoreType.DMA,
        ],
        compiler_params=pltpu.CompilerParams(use_tc_tiling_on_sc=False),
    )(_gather_body)
    return k(table_flat, idx_flat)


def _mlp_body(x_ref, w1, b1, w2, b2, w3, b3, w4, b4, w5, b5, w6, b6, o_ref):
    h = x_ref[...]
    h = jnp.maximum(jnp.dot(h, w1[...], preferred_element_type=jnp.float32) + b1[...], 0.0)
    h = jnp.maximum(jnp.dot(h, w2[...], preferred_element_type=jnp.float32) + b2[...], 0.0)
    h = jnp.maximum(jnp.dot(h, w3[...], preferred_element_type=jnp.float32) + b3[...], 0.0)
    h = jnp.maximum(jnp.dot(h, w4[...], preferred_element_type=jnp.float32) + b4[...], 0.0)
    h = jnp.maximum(jnp.dot(h, w5[...], preferred_element_type=jnp.float32) + b5[...], 0.0)
    z = jnp.dot(h, w6[...], preferred_element_type=jnp.float32) + b6[...]
    o_ref[...] = jax.nn.sigmoid(z)


_BB = 512


def _tc_mlp(x, w1, b1, w2, b2, w3, b3, w4, b4, w5, b5, w6, b6):
    full = lambda a: pl.BlockSpec(a.shape, lambda i: (0, 0))
    return pl.pallas_call(
        _mlp_body,
        grid=(B // _BB,),
        in_specs=[pl.BlockSpec((_BB, CIN), lambda i: (i, 0))]
        + [full(a) for a in (w1, b1, w2, b2, w3, b3, w4, b4, w5, b5, w6, b6)],
        out_specs=pl.BlockSpec((_BB, 1), lambda i: (i, 0)),
        out_shape=jax.ShapeDtypeStruct((B, 1), jnp.float32),
    )(x, w1, b1, w2, b2, w3, b3, w4, b4, w5, b5, w6, b6)


def kernel(indices, emb_tables, W1, b1, W2, b2, W3, b3, W4, b4, W5, b5, W6, b6):
    table_flat = jnp.pad(emb_tables, ((0, 0), (0, 0), (0, DP - D))).reshape(F * V, DP)
    idx_flat = indices.T.astype(jnp.int32).reshape(F * B)
    w1p = jnp.pad(W1.reshape(F, D, -1), ((0, 0), (0, DP - D), (0, 0))).reshape(CIN, -1)

    x = _sc_gather(table_flat, idx_flat)

    args = (w1p, b1, W2, b2, W3, b3, W4, b4, W5, b5, W6, b6)
    args = tuple(a if a.ndim == 2 else a.reshape(1, -1) for a in args)
    return _tc_mlp(x, *args)


# speedup vs baseline: 21.4164x; 1.2697x over previous
import functools

import jax
import jax.numpy as jnp
from jax import lax
from jax.experimental import pallas as pl
from jax.experimental.pallas import tpu as pltpu
from jax.experimental.pallas import tpu_sc as plsc

B = 4096
F = 26
V = 1001
D = 29
DP = 32
CIN = F * DP

_NW = 32
_BPW = B // _NW


_NPW = _BPW * F


def _gather_body(table_hbm, idx_hbm, off_hbm, out_hbm, idx_v, off_v, rows_v, sem):
    c = lax.axis_index("c")
    s = lax.axis_index("s")
    wid = s * 2 + c
    p0 = wid * _NPW

    pltpu.sync_copy(idx_hbm.at[pl.ds(p0, _NPW)], idx_v)
    pltpu.sync_copy(off_hbm.at[pl.ds(p0, _NPW)], off_v)
    for i in range(_NPW // 16):
        sl = pl.ds(i * 16, 16)
        idx_v[sl] = idx_v[sl] + off_v[sl]
    pltpu.async_copy(table_hbm.at[idx_v], rows_v, sem).wait()
    pltpu.sync_copy(rows_v, out_hbm.at[pl.ds(p0, _NPW), :])


def _sc_gather(table_flat, idx_flat, off_flat):
    mesh = plsc.VectorSubcoreMesh(core_axis_name="c", subcore_axis_name="s")
    k = functools.partial(
        pl.kernel,
        mesh=mesh,
        out_type=jax.ShapeDtypeStruct((B * F, DP), jnp.float32),
        scratch_types=[
            pltpu.VMEM((_NPW,), jnp.int32),
            pltpu.VMEM((_NPW,), jnp.int32),
            pltpu.VMEM((_NPW, DP), jnp.float32),
            pltpu.SemaphoreType.DMA,
        ],
        compiler_params=pltpu.CompilerParams(use_tc_tiling_on_sc=False),
    )(_gather_body)
    return k(table_flat, idx_flat, off_flat)


def _mlp_body(x_ref, w1, b1, w2, b2, w3, b3, w4, b4, w5, b5, w6, b6, o_ref):
    h = x_ref[...]
    h = jnp.maximum(jnp.dot(h, w1[...], preferred_element_type=jnp.float32) + b1[...], 0.0)
    h = jnp.maximum(jnp.dot(h, w2[...], preferred_element_type=jnp.float32) + b2[...], 0.0)
    h = jnp.maximum(jnp.dot(h, w3[...], preferred_element_type=jnp.float32) + b3[...], 0.0)
    h = jnp.maximum(jnp.dot(h, w4[...], preferred_element_type=jnp.float32) + b4[...], 0.0)
    h = jnp.maximum(jnp.dot(h, w5[...], preferred_element_type=jnp.float32) + b5[...], 0.0)
    z = jnp.dot(h, w6[...], preferred_element_type=jnp.float32) + b6[...]
    o_ref[...] = jax.nn.sigmoid(z)


_BB = 512


def _tc_mlp(x, w1, b1, w2, b2, w3, b3, w4, b4, w5, b5, w6, b6):
    full = lambda a: pl.BlockSpec(a.shape, lambda i: (0, 0))
    return pl.pallas_call(
        _mlp_body,
        grid=(B // _BB,),
        in_specs=[pl.BlockSpec((_BB, CIN), lambda i: (i, 0))]
        + [full(a) for a in (w1, b1, w2, b2, w3, b3, w4, b4, w5, b5, w6, b6)],
        out_specs=pl.BlockSpec((_BB, 1), lambda i: (i, 0)),
        out_shape=jax.ShapeDtypeStruct((B, 1), jnp.float32),
    )(x, w1, b1, w2, b2, w3, b3, w4, b4, w5, b5, w6, b6)


def kernel(indices, emb_tables, W1, b1, W2, b2, W3, b3, W4, b4, W5, b5, W6, b6):
    table_flat = jnp.pad(emb_tables, ((0, 0), (0, 0), (0, DP - D))).reshape(F * V, DP)
    idx_flat = indices.astype(jnp.int32).reshape(B * F)
    off_flat = jnp.tile(jnp.arange(F, dtype=jnp.int32) * V, B)
    w1p = jnp.pad(W1.reshape(F, D, -1), ((0, 0), (0, DP - D), (0, 0))).reshape(CIN, -1)

    x = _sc_gather(table_flat, idx_flat, off_flat).reshape(B, CIN)

    args = (w1p, b1, W2, b2, W3, b3, W4, b4, W5, b5, W6, b6)
    args = tuple(a if a.ndim == 2 else a.reshape(1, -1) for a in args)
    return _tc_mlp(x, *args)


# speedup vs baseline: 22.4974x; 1.0505x over previous
import functools

import jax
import jax.numpy as jnp
from jax import lax
from jax.experimental import pallas as pl
from jax.experimental.pallas import tpu as pltpu
from jax.experimental.pallas import tpu_sc as plsc

B = 4096
F = 26
V = 1001
D = 29
DP = 32
CIN = F * DP

_NW = 32
_BPW = B // _NW
_NPW = _BPW * F


def _gather_body(table_hbm, idx_hbm, out_hbm, idx_v, rows_v, sem, wsem):
    c = lax.axis_index("c")
    s = lax.axis_index("s")
    wid = s * 2 + c
    p0 = wid * _NPW
    b0 = wid * _BPW

    pltpu.sync_copy(idx_hbm.at[pl.ds(p0, _NPW)], idx_v)
    for i in range(_NPW // 16):
        f = i // (_BPW // 16)
        sl = pl.ds(i * 16, 16)
        idx_v[sl] = idx_v[sl] + f * V
    pltpu.async_copy(table_hbm.at[idx_v], rows_v, sem).wait()
    handles = [
        pltpu.async_copy(
            rows_v.at[pl.ds(f * _BPW, _BPW), :],
            out_hbm.at[pl.ds(b0, _BPW), pl.ds(f * DP, DP)],
            wsem,
        )
        for f in range(F)
    ]
    for h in handles:
        h.wait()


def _sc_gather(table_flat, idx_perm):
    mesh = plsc.VectorSubcoreMesh(core_axis_name="c", subcore_axis_name="s")
    k = functools.partial(
        pl.kernel,
        mesh=mesh,
        out_type=jax.ShapeDtypeStruct((B, CIN), jnp.float32),
        scratch_types=[
            pltpu.VMEM((_NPW,), jnp.int32),
            pltpu.VMEM((_NPW, DP), jnp.float32),
            pltpu.SemaphoreType.DMA,
            pltpu.SemaphoreType.DMA,
        ],
        compiler_params=pltpu.CompilerParams(use_tc_tiling_on_sc=False),
    )(_gather_body)
    return k(table_flat, idx_perm)


def _mlp_body(x_ref, w1, b1, w2, b2, w3, b3, w4, b4, w5, b5, w6, b6, o_ref):
    h = x_ref[...]
    h = jnp.maximum(jnp.dot(h, w1[...], preferred_element_type=jnp.float32) + b1[...], 0.0)
    h = jnp.maximum(jnp.dot(h, w2[...], preferred_element_type=jnp.float32) + b2[...], 0.0)
    h = jnp.maximum(jnp.dot(h, w3[...], preferred_element_type=jnp.float32) + b3[...], 0.0)
    h = jnp.maximum(jnp.dot(h, w4[...], preferred_element_type=jnp.float32) + b4[...], 0.0)
    h = jnp.maximum(jnp.dot(h, w5[...], preferred_element_type=jnp.float32) + b5[...], 0.0)
    z = jnp.dot(h, w6[...], preferred_element_type=jnp.float32) + b6[...]
    o_ref[...] = jax.nn.sigmoid(z)


_BB = 512


def _tc_mlp(x, w1, b1, w2, b2, w3, b3, w4, b4, w5, b5, w6, b6):
    full = lambda a: pl.BlockSpec(a.shape, lambda i: (0, 0))
    return pl.pallas_call(
        _mlp_body,
        grid=(B // _BB,),
        in_specs=[pl.BlockSpec((_BB, CIN), lambda i: (i, 0))]
        + [full(a) for a in (w1, b1, w2, b2, w3, b3, w4, b4, w5, b5, w6, b6)],
        out_specs=pl.BlockSpec((_BB, 1), lambda i: (i, 0)),
        out_shape=jax.ShapeDtypeStruct((B, 1), jnp.float32),
    )(x, w1, b1, w2, b2, w3, b3, w4, b4, w5, b5, w6, b6)


def kernel(indices, emb_tables, W1, b1, W2, b2, W3, b3, W4, b4, W5, b5, W6, b6):
    table_flat = jnp.pad(emb_tables, ((0, 0), (0, 0), (0, DP - D))).reshape(F * V, DP)
    idx_perm = (
        indices.astype(jnp.int32).reshape(_NW, _BPW, F).transpose(0, 2, 1).reshape(-1)
    )
    w1p = jnp.pad(W1.reshape(F, D, -1), ((0, 0), (0, DP - D), (0, 0))).reshape(CIN, -1)

    x = _sc_gather(table_flat, idx_perm)

    args = (w1p, b1, W2, b2, W3, b3, W4, b4, W5, b5, W6, b6)
    args = tuple(a if a.ndim == 2 else a.reshape(1, -1) for a in args)
    return _tc_mlp(x, *args)
